# Initial kernel scaffold; baseline (speedup 1.0000x reference)
import functools

import jax
import jax.numpy as jnp
from jax import lax
from jax.experimental import pallas as pl
from jax.experimental.pallas import tpu as pltpu
from jax.experimental.pallas import tpu_sc as plsc

N_CORES = 2
N_SUBCORES = 16
N_TILES = N_CORES * N_SUBCORES
K = 80
M_BLK = 1000


def _sc_mesh():
    return plsc.VectorSubcoreMesh(core_axis_name="c", subcore_axis_name="s")


def _make_deg_kernel(E, N):
    ept = E // N_TILES
    nch = ept // K
    rps = N // N_SUBCORES

    @functools.partial(
        pl.kernel,
        out_type=jax.ShapeDtypeStruct((N_CORES, N, 16), jnp.float32),
        mesh=_sc_mesh(),
        scratch_types=[
            pltpu.VMEM((2, K), jnp.int32),
            pltpu.VMEM((K, 16), jnp.float32),
            pltpu.VMEM_SHARED((N, 16), jnp.float32),
        ],
    )
    def deg_kernel(dst_hbm, zero_hbm, ones_hbm, out_hbm, didx, ones_v, acc_sh):
        c = lax.axis_index("c")
        s = lax.axis_index("s")
        wid = c * N_SUBCORES + s
        pltpu.sync_copy(ones_hbm, ones_v)
        pltpu.sync_copy(zero_hbm.at[pl.ds(s * rps, rps)],
                        acc_sh.at[pl.ds(s * rps, rps)])
        plsc.subcore_barrier()
        base = wid * ept

        @pl.loop(0, nch)
        def _(ch):
            off = base + ch * K
            pltpu.sync_copy(dst_hbm.at[pl.ds(off, K)], didx.at[0])
            pltpu.sync_copy(ones_v, acc_sh.at[didx.at[0]], add=True)

        plsc.subcore_barrier()
        pltpu.sync_copy(acc_sh.at[pl.ds(s * rps, rps)],
                        out_hbm.at[c, pl.ds(s * rps, rps)])

    return deg_kernel


def _make_agg_kernel(E, N, D):
    ept = E // N_TILES
    nch = ept // K
    rps = N // N_SUBCORES

    @functools.partial(
        pl.kernel,
        out_type=jax.ShapeDtypeStruct((N_CORES, N, D), jnp.float32),
        mesh=_sc_mesh(),
        scratch_types=[
            pltpu.VMEM((2, K), jnp.int32),
            pltpu.VMEM((2, K), jnp.int32),
            pltpu.VMEM((K, D), jnp.float32),
            pltpu.VMEM_SHARED((N, D), jnp.float32),
            pltpu.SemaphoreType.DMA,
        ],
    )
    def agg_kernel(g_hbm, src_hbm, dst_hbm, zero_hbm, out_hbm,
                   sidx, didx, rows, acc_sh, sem):
        c = lax.axis_index("c")
        s = lax.axis_index("s")
        wid = c * N_SUBCORES + s
        pltpu.sync_copy(zero_hbm.at[pl.ds(s * rps, rps)],
                        acc_sh.at[pl.ds(s * rps, rps)])
        plsc.subcore_barrier()
        base = wid * ept

        @pl.loop(0, nch)
        def _(ch):
            off = base + ch * K
            pltpu.sync_copy(src_hbm.at[pl.ds(off, K)], sidx.at[0])
            pltpu.sync_copy(dst_hbm.at[pl.ds(off, K)], didx.at[0])
            pltpu.async_copy(g_hbm.at[sidx.at[0]], rows, sem).wait()
            pltpu.sync_copy(rows, acc_sh.at[didx.at[0]], add=True)

        plsc.subcore_barrier()
        pltpu.sync_copy(acc_sh.at[pl.ds(s * rps, rps)],
                        out_hbm.at[c, pl.ds(s * rps, rps)])

    return agg_kernel


def _dot(a, b):
    return lax.dot_general(a, b, (((1,), (0,)), ((), ())),
                           precision=lax.Precision.HIGHEST,
                           preferred_element_type=jnp.float32)


def _matmul_body(x_ref, w_ref, o_ref):
    o_ref[...] = _dot(x_ref[...], w_ref[...])


def _scale_body(deg_ref, h_ref, g_ref, dinv_ref):
    degsum = deg_ref[0] + deg_ref[1]
    dinv = lax.rsqrt(degsum[:, 0:1] + 1.0)
    dinv_ref[...] = dinv
    g_ref[...] = h_ref[...] * dinv


def _epi1_body(agg_ref, g1_ref, dinv_ref, b1_ref, w2_ref, g2_ref):
    dinv = dinv_ref[...]
    pre = dinv * (agg_ref[0] + agg_ref[1] + g1_ref[...]) + b1_ref[...]
    r = jnp.maximum(pre, 0.0)
    g2_ref[...] = _dot(r, w2_ref[...]) * dinv


def _epi2_body(agg_ref, g2_ref, dinv_ref, b2_ref, o_ref):
    o_ref[...] = (dinv_ref[...] * (agg_ref[0] + agg_ref[1] + g2_ref[...])
                  + b2_ref[...])


def kernel(x, edge_index, W1, b1, W2, b2):
    N, FEAT = x.shape
    HID = W1.shape[1]
    OUT = W2.shape[1]
    E = edge_index.shape[1]
    grid = (N // M_BLK,)

    src = edge_index[0]
    dst = edge_index[1]
    z16 = jnp.zeros((N, 16), jnp.float32)
    ones = jnp.ones((K, 16), jnp.float32)
    zH = jnp.zeros((N, HID), jnp.float32)
    zO = jnp.zeros((N, OUT), jnp.float32)

    deg = _make_deg_kernel(E, N)(dst, z16, ones)

    h1 = pl.pallas_call(
        _matmul_body,
        grid=grid,
        in_specs=[pl.BlockSpec((M_BLK, FEAT), lambda i: (i, 0)),
                  pl.BlockSpec((FEAT, HID), lambda i: (0, 0))],
        out_specs=pl.BlockSpec((M_BLK, HID), lambda i: (i, 0)),
        out_shape=jax.ShapeDtypeStruct((N, HID), jnp.float32),
    )(x, W1)

    g1, dinv = pl.pallas_call(
        _scale_body,
        grid=grid,
        in_specs=[pl.BlockSpec((N_CORES, M_BLK, 16), lambda i: (0, i, 0)),
                  pl.BlockSpec((M_BLK, HID), lambda i: (i, 0))],
        out_specs=[pl.BlockSpec((M_BLK, HID), lambda i: (i, 0)),
                   pl.BlockSpec((M_BLK, 1), lambda i: (i, 0))],
        out_shape=[jax.ShapeDtypeStruct((N, HID), jnp.float32),
                   jax.ShapeDtypeStruct((N, 1), jnp.float32)],
    )(deg, h1)

    agg1 = _make_agg_kernel(E, N, HID)(g1, src, dst, zH)

    g2 = pl.pallas_call(
        _epi1_body,
        grid=grid,
        in_specs=[pl.BlockSpec((N_CORES, M_BLK, HID), lambda i: (0, i, 0)),
                  pl.BlockSpec((M_BLK, HID), lambda i: (i, 0)),
                  pl.BlockSpec((M_BLK, 1), lambda i: (i, 0)),
                  pl.BlockSpec((1, HID), lambda i: (0, 0)),
                  pl.BlockSpec((HID, OUT), lambda i: (0, 0))],
        out_specs=pl.BlockSpec((M_BLK, OUT), lambda i: (i, 0)),
        out_shape=jax.ShapeDtypeStruct((N, OUT), jnp.float32),
    )(agg1, g1, dinv, b1.reshape(1, HID), W2)

    agg2 = _make_agg_kernel(E, N, OUT)(g2, src, dst, zO)

    out = pl.pallas_call(
        _epi2_body,
        grid=grid,
        in_specs=[pl.BlockSpec((N_CORES, M_BLK, OUT), lambda i: (0, i, 0)),
                  pl.BlockSpec((M_BLK, OUT), lambda i: (i, 0)),
                  pl.BlockSpec((M_BLK, 1), lambda i: (i, 0)),
                  pl.BlockSpec((1, OUT), lambda i: (0, 0))],
        out_specs=pl.BlockSpec((M_BLK, OUT), lambda i: (i, 0)),
        out_shape=jax.ShapeDtypeStruct((N, OUT), jnp.float32),
    )(agg2, g2, dinv, b2.reshape(1, OUT))

    return out


# speedup vs baseline: 13.5377x; 13.5377x over previous
import functools

import jax
import jax.numpy as jnp
from jax import lax
from jax.experimental import pallas as pl
from jax.experimental.pallas import tpu as pltpu
from jax.experimental.pallas import tpu_sc as plsc

N_CORES = 2
N_SUBCORES = 16
N_TILES = N_CORES * N_SUBCORES
K = 80
M_BLK = 1000


def _sc_mesh():
    return plsc.VectorSubcoreMesh(core_axis_name="c", subcore_axis_name="s")


def _row_partition(N):
    rps = (N // 8 // N_SUBCORES) * 8
    tail = N - N_SUBCORES * rps
    return rps, tail


def _striped_rows_copy(src, dst, s, N):
    rps, tail = _row_partition(N)
    pltpu.sync_copy(src.at[pl.ds(s * rps, rps)], dst.at[pl.ds(s * rps, rps)])
    if tail:
        @pl.when(s == N_SUBCORES - 1)
        def _():
            pltpu.sync_copy(src.at[pl.ds(N_SUBCORES * rps, tail)],
                            dst.at[pl.ds(N_SUBCORES * rps, tail)])


def _make_deg_kernel(E, N):
    ept = E // N_TILES
    nch = ept // K

    @functools.partial(
        pl.kernel,
        out_type=jax.ShapeDtypeStruct((N_CORES, N, 16), jnp.float32),
        mesh=_sc_mesh(),
        compiler_params=pltpu.CompilerParams(use_tc_tiling_on_sc=False),
        scratch_types=[
            pltpu.VMEM((2, K), jnp.int32),
            pltpu.VMEM((K, 16), jnp.float32),
            pltpu.VMEM_SHARED((N, 16), jnp.float32),
        ],
    )
    def deg_kernel(dst_hbm, zero_hbm, ones_hbm, out_hbm, didx, ones_v, acc_sh):
        c = lax.axis_index("c")
        s = lax.axis_index("s")
        wid = c * N_SUBCORES + s
        pltpu.sync_copy(ones_hbm, ones_v)
        _striped_rows_copy(zero_hbm, acc_sh, s, N)
        plsc.subcore_barrier()
        base = wid * ept

        @pl.loop(0, nch)
        def _(ch):
            off = base + ch * K
            pltpu.sync_copy(dst_hbm.at[pl.ds(off, K)], didx.at[0])
            pltpu.sync_copy(ones_v, acc_sh.at[didx.at[0]], add=True)

        plsc.subcore_barrier()
        _striped_rows_copy(acc_sh, out_hbm.at[c], s, N)

    return deg_kernel


def _make_agg_kernel(E, N, D):
    ept = E // N_TILES
    nch = ept // K

    @functools.partial(
        pl.kernel,
        out_type=jax.ShapeDtypeStruct((N_CORES, N, D), jnp.float32),
        mesh=_sc_mesh(),
        compiler_params=pltpu.CompilerParams(use_tc_tiling_on_sc=False),
        scratch_types=[
            pltpu.VMEM((2, K), jnp.int32),
            pltpu.VMEM((2, K), jnp.int32),
            pltpu.VMEM((K, D), jnp.float32),
            pltpu.VMEM_SHARED((N, D), jnp.float32),
            pltpu.SemaphoreType.DMA,
        ],
    )
    def agg_kernel(g_hbm, src_hbm, dst_hbm, zero_hbm, out_hbm,
                   sidx, didx, rows, acc_sh, sem):
        c = lax.axis_index("c")
        s = lax.axis_index("s")
        wid = c * N_SUBCORES + s
        _striped_rows_copy(zero_hbm, acc_sh, s, N)
        plsc.subcore_barrier()
        base = wid * ept

        @pl.loop(0, nch)
        def _(ch):
            off = base + ch * K
            pltpu.sync_copy(src_hbm.at[pl.ds(off, K)], sidx.at[0])
            pltpu.sync_copy(dst_hbm.at[pl.ds(off, K)], didx.at[0])
            pltpu.async_copy(g_hbm.at[sidx.at[0]], rows, sem).wait()
            pltpu.sync_copy(rows, acc_sh.at[didx.at[0]], add=True)

        plsc.subcore_barrier()
        _striped_rows_copy(acc_sh, out_hbm.at[c], s, N)

    return agg_kernel


def _dot(a, b):
    return lax.dot_general(a, b, (((1,), (0,)), ((), ())),
                           precision=lax.Precision.HIGHEST,
                           preferred_element_type=jnp.float32)


def _matmul_body(x_ref, w_ref, o_ref):
    o_ref[...] = _dot(x_ref[...], w_ref[...])


def _scale_body(deg_ref, h_ref, g_ref, dinv_ref):
    degsum = deg_ref[0] + deg_ref[1]
    dinv = lax.rsqrt(degsum[:, 0:1] + 1.0)
    dinv_ref[...] = dinv
    g_ref[...] = h_ref[...] * dinv


def _epi1_body(agg_ref, g1_ref, dinv_ref, b1_ref, w2_ref, g2_ref):
    dinv = dinv_ref[...]
    pre = dinv * (agg_ref[0] + agg_ref[1] + g1_ref[...]) + b1_ref[...]
    r = jnp.maximum(pre, 0.0)
    g2_ref[...] = _dot(r, w2_ref[...]) * dinv


def _epi2_body(agg_ref, g2_ref, dinv_ref, b2_ref, o_ref):
    o_ref[...] = (dinv_ref[...] * (agg_ref[0] + agg_ref[1] + g2_ref[...])
                  + b2_ref[...])


def kernel(x, edge_index, W1, b1, W2, b2):
    N, FEAT = x.shape
    HID = W1.shape[1]
    OUT = W2.shape[1]
    E = edge_index.shape[1]
    grid = (N // M_BLK,)

    src = edge_index[0]
    dst = edge_index[1]
    z16 = jnp.zeros((N, 16), jnp.float32)
    ones = jnp.ones((K, 16), jnp.float32)
    zH = jnp.zeros((N, HID), jnp.float32)
    zO = jnp.zeros((N, OUT), jnp.float32)

    deg = _make_deg_kernel(E, N)(dst, z16, ones)

    h1 = pl.pallas_call(
        _matmul_body,
        grid=grid,
        in_specs=[pl.BlockSpec((M_BLK, FEAT), lambda i: (i, 0)),
                  pl.BlockSpec((FEAT, HID), lambda i: (0, 0))],
        out_specs=pl.BlockSpec((M_BLK, HID), lambda i: (i, 0)),
        out_shape=jax.ShapeDtypeStruct((N, HID), jnp.float32),
    )(x, W1)

    g1, dinv = pl.pallas_call(
        _scale_body,
        grid=grid,
        in_specs=[pl.BlockSpec((N_CORES, M_BLK, 16), lambda i: (0, i, 0)),
                  pl.BlockSpec((M_BLK, HID), lambda i: (i, 0))],
        out_specs=[pl.BlockSpec((M_BLK, HID), lambda i: (i, 0)),
                   pl.BlockSpec((M_BLK, 1), lambda i: (i, 0))],
        out_shape=[jax.ShapeDtypeStruct((N, HID), jnp.float32),
                   jax.ShapeDtypeStruct((N, 1), jnp.float32)],
    )(deg, h1)

    agg1 = _make_agg_kernel(E, N, HID)(g1, src, dst, zH)

    g2 = pl.pallas_call(
        _epi1_body,
        grid=grid,
        in_specs=[pl.BlockSpec((N_CORES, M_BLK, HID), lambda i: (0, i, 0)),
                  pl.BlockSpec((M_BLK, HID), lambda i: (i, 0)),
                  pl.BlockSpec((M_BLK, 1), lambda i: (i, 0)),
                  pl.BlockSpec((1, HID), lambda i: (0, 0)),
                  pl.BlockSpec((HID, OUT), lambda i: (0, 0))],
        out_specs=pl.BlockSpec((M_BLK, OUT), lambda i: (i, 0)),
        out_shape=jax.ShapeDtypeStruct((N, OUT), jnp.float32),
    )(agg1, g1, dinv, b1.reshape(1, HID), W2)

    agg2 = _make_agg_kernel(E, N, OUT)(g2, src, dst, zO)

    out = pl.pallas_call(
        _epi2_body,
        grid=grid,
        in_specs=[pl.BlockSpec((N_CORES, M_BLK, OUT), lambda i: (0, i, 0)),
                  pl.BlockSpec((M_BLK, OUT), lambda i: (i, 0)),
                  pl.BlockSpec((M_BLK, 1), lambda i: (i, 0)),
                  pl.BlockSpec((1, OUT), lambda i: (0, 0))],
        out_specs=pl.BlockSpec((M_BLK, OUT), lambda i: (i, 0)),
        out_shape=jax.ShapeDtypeStruct((N, OUT), jnp.float32),
    )(agg2, g2, dinv, b2.reshape(1, OUT))

    return out


# speedup vs baseline: 32.4086x; 2.3940x over previous
import functools

import jax
import jax.numpy as jnp
from jax import lax
from jax.experimental import pallas as pl
from jax.experimental.pallas import tpu as pltpu
from jax.experimental.pallas import tpu_sc as plsc

N_CORES = 2
N_SUBCORES = 16
N_TILES = N_CORES * N_SUBCORES
K = 80
M_BLK = 1000


def _sc_mesh():
    return plsc.VectorSubcoreMesh(core_axis_name="c", subcore_axis_name="s")


def _row_partition(N):
    rps = (N // 8 // N_SUBCORES) * 8
    tail = N - N_SUBCORES * rps
    return rps, tail


def _striped_rows_copy(src, dst, s, N):
    rps, tail = _row_partition(N)
    pltpu.sync_copy(src.at[pl.ds(s * rps, rps)], dst.at[pl.ds(s * rps, rps)])
    if tail:
        @pl.when(s == N_SUBCORES - 1)
        def _():
            pltpu.sync_copy(src.at[pl.ds(N_SUBCORES * rps, tail)],
                            dst.at[pl.ds(N_SUBCORES * rps, tail)])


NBUF = 5


def _make_deg_kernel(E, N):
    ept = E // N_TILES
    nch = ept // K

    @functools.partial(
        pl.kernel,
        out_type=jax.ShapeDtypeStruct((N_CORES, N, 16), jnp.float32),
        mesh=_sc_mesh(),
        compiler_params=pltpu.CompilerParams(use_tc_tiling_on_sc=False),
        scratch_types=[
            pltpu.VMEM((E // K // N_TILES, K), jnp.int32),
            pltpu.VMEM((K, 16), jnp.float32),
            pltpu.VMEM_SHARED((N, 16), jnp.float32),
            pltpu.SemaphoreType.DMA,
        ],
    )
    def deg_kernel(dst2_hbm, zero_hbm, ones_hbm, out_hbm,
                   didx, ones_v, acc_sh, ssem):
        c = lax.axis_index("c")
        s = lax.axis_index("s")
        wid = c * N_SUBCORES + s
        pltpu.sync_copy(ones_hbm, ones_v)
        pltpu.sync_copy(dst2_hbm.at[pl.ds(wid * nch, nch)], didx)
        _striped_rows_copy(zero_hbm, acc_sh, s, N)
        plsc.subcore_barrier()

        @pl.loop(0, nch, step=NBUF)
        def _(c0):
            for b in range(NBUF):
                pltpu.async_copy(ones_v, acc_sh.at[didx.at[c0 + b]], ssem,
                                 add=True)
            for b in range(NBUF):
                pltpu.make_async_copy(ones_v, acc_sh.at[didx.at[0]],
                                      ssem).wait()

        plsc.subcore_barrier()
        _striped_rows_copy(acc_sh, out_hbm.at[c], s, N)

    return deg_kernel


def _make_agg_kernel(E, N, D, feature_split):
    n_edge_workers = N_SUBCORES if feature_split else N_TILES
    ept = E // n_edge_workers
    nch = ept // K
    assert nch % NBUF == 0 and ept % K == 0
    g_shape = (N_CORES, N, D) if feature_split else (N, D)

    @functools.partial(
        pl.kernel,
        out_type=jax.ShapeDtypeStruct((N_CORES, N, D), jnp.float32),
        mesh=_sc_mesh(),
        compiler_params=pltpu.CompilerParams(use_tc_tiling_on_sc=False),
        scratch_types=[
            pltpu.VMEM((nch, K), jnp.int32),
            pltpu.VMEM((nch, K), jnp.int32),
            pltpu.VMEM((NBUF, K, D), jnp.float32),
            pltpu.VMEM_SHARED((N, D), jnp.float32),
            pltpu.SemaphoreType.DMA((NBUF,)),
            pltpu.SemaphoreType.DMA((NBUF,)),
        ],
    )
    def agg_kernel(g_hbm, src2_hbm, dst2_hbm, zero_hbm, out_hbm,
                   sidx, didx, rows, acc_sh, gsem, ssem):
        c = lax.axis_index("c")
        s = lax.axis_index("s")
        wid = s if feature_split else c * N_SUBCORES + s
        gsrc = g_hbm.at[c] if feature_split else g_hbm
        pltpu.sync_copy(src2_hbm.at[pl.ds(wid * nch, nch)], sidx)
        pltpu.sync_copy(dst2_hbm.at[pl.ds(wid * nch, nch)], didx)
        _striped_rows_copy(zero_hbm, acc_sh, s, N)
        plsc.subcore_barrier()

        @pl.loop(0, nch, step=NBUF)
        def _(c0):
            for b in range(NBUF):
                @pl.when(c0 > 0)
                def _():
                    pltpu.make_async_copy(rows.at[b],
                                          acc_sh.at[didx.at[0]],
                                          ssem.at[b]).wait()
                pltpu.async_copy(gsrc.at[sidx.at[c0 + b]], rows.at[b],
                                 gsem.at[b])
            for b in range(NBUF):
                pltpu.make_async_copy(gsrc.at[sidx.at[0]], rows.at[b],
                                      gsem.at[b]).wait()
                pltpu.async_copy(rows.at[b], acc_sh.at[didx.at[c0 + b]],
                                 ssem.at[b], add=True)

        for b in range(NBUF):
            pltpu.make_async_copy(rows.at[b], acc_sh.at[didx.at[0]],
                                  ssem.at[b]).wait()
        plsc.subcore_barrier()
        _striped_rows_copy(acc_sh, out_hbm.at[c], s, N)

    return agg_kernel


def _dot(a, b):
    return lax.dot_general(a, b, (((1,), (0,)), ((), ())),
                           precision=lax.Precision.HIGHEST,
                           preferred_element_type=jnp.float32)


def _matmul_body(x_ref, w_ref, o_ref):
    o_ref[...] = _dot(x_ref[...], w_ref[...])


def _scale_body(deg_ref, h_ref, g_ref, dinv_ref):
    degsum = deg_ref[0] + deg_ref[1]
    dinv = lax.rsqrt(degsum[:, 0:1] + 1.0)
    dinv_ref[...] = dinv
    g = h_ref[...] * dinv
    half = g.shape[1] // 2
    g_ref[0] = g[:, :half]
    g_ref[1] = g[:, half:]


def _epi1_body(agg_ref, g1_ref, dinv_ref, b1_ref, w2_ref, g2_ref):
    dinv = dinv_ref[...]
    summed = jnp.concatenate([agg_ref[0] + g1_ref[0],
                              agg_ref[1] + g1_ref[1]], axis=1)
    pre = dinv * summed + b1_ref[...]
    r = jnp.maximum(pre, 0.0)
    g2_ref[...] = _dot(r, w2_ref[...]) * dinv


def _epi2_body(agg_ref, g2_ref, dinv_ref, b2_ref, o_ref):
    o_ref[...] = (dinv_ref[...] * (agg_ref[0] + agg_ref[1] + g2_ref[...])
                  + b2_ref[...])


def kernel(x, edge_index, W1, b1, W2, b2):
    N, FEAT = x.shape
    HID = W1.shape[1]
    OUT = W2.shape[1]
    E = edge_index.shape[1]
    grid = (N // M_BLK,)

    HALF = HID // N_CORES
    src2 = edge_index[0].reshape(E // K, K)
    dst2 = edge_index[1].reshape(E // K, K)
    z16 = jnp.zeros((N, 16), jnp.float32)
    ones = jnp.ones((K, 16), jnp.float32)
    zH = jnp.zeros((N, HALF), jnp.float32)
    zO = jnp.zeros((N, OUT), jnp.float32)

    deg = _make_deg_kernel(E, N)(dst2, z16, ones)

    h1 = pl.pallas_call(
        _matmul_body,
        grid=grid,
        in_specs=[pl.BlockSpec((M_BLK, FEAT), lambda i: (i, 0)),
                  pl.BlockSpec((FEAT, HID), lambda i: (0, 0))],
        out_specs=pl.BlockSpec((M_BLK, HID), lambda i: (i, 0)),
        out_shape=jax.ShapeDtypeStruct((N, HID), jnp.float32),
    )(x, W1)

    g1, dinv = pl.pallas_call(
        _scale_body,
        grid=grid,
        in_specs=[pl.BlockSpec((N_CORES, M_BLK, 16), lambda i: (0, i, 0)),
                  pl.BlockSpec((M_BLK, HID), lambda i: (i, 0))],
        out_specs=[pl.BlockSpec((N_CORES, M_BLK, HALF), lambda i: (0, i, 0)),
                   pl.BlockSpec((M_BLK, 1), lambda i: (i, 0))],
        out_shape=[jax.ShapeDtypeStruct((N_CORES, N, HALF), jnp.float32),
                   jax.ShapeDtypeStruct((N, 1), jnp.float32)],
    )(deg, h1)

    agg1 = _make_agg_kernel(E, N, HALF, True)(g1, src2, dst2, zH)

    g2 = pl.pallas_call(
        _epi1_body,
        grid=grid,
        in_specs=[pl.BlockSpec((N_CORES, M_BLK, HALF), lambda i: (0, i, 0)),
                  pl.BlockSpec((N_CORES, M_BLK, HALF), lambda i: (0, i, 0)),
                  pl.BlockSpec((M_BLK, 1), lambda i: (i, 0)),
                  pl.BlockSpec((1, HID), lambda i: (0, 0)),
                  pl.BlockSpec((HID, OUT), lambda i: (0, 0))],
        out_specs=pl.BlockSpec((M_BLK, OUT), lambda i: (i, 0)),
        out_shape=jax.ShapeDtypeStruct((N, OUT), jnp.float32),
    )(agg1, g1, dinv, b1.reshape(1, HID), W2)

    agg2 = _make_agg_kernel(E, N, OUT, False)(g2, src2, dst2, zO)

    out = pl.pallas_call(
        _epi2_body,
        grid=grid,
        in_specs=[pl.BlockSpec((N_CORES, M_BLK, OUT), lambda i: (0, i, 0)),
                  pl.BlockSpec((M_BLK, OUT), lambda i: (i, 0)),
                  pl.BlockSpec((M_BLK, 1), lambda i: (i, 0)),
                  pl.BlockSpec((1, OUT), lambda i: (0, 0))],
        out_specs=pl.BlockSpec((M_BLK, OUT), lambda i: (i, 0)),
        out_shape=jax.ShapeDtypeStruct((N, OUT), jnp.float32),
    )(agg2, g2, dinv, b2.reshape(1, OUT))

    return out


# speedup vs baseline: 34.4526x; 1.0631x over previous
import functools

import jax
import jax.numpy as jnp
from jax import lax
from jax.experimental import pallas as pl
from jax.experimental.pallas import tpu as pltpu
from jax.experimental.pallas import tpu_sc as plsc

N_CORES = 2
N_SUBCORES = 16
N_TILES = N_CORES * N_SUBCORES
K = 80
M_BLK = 2000


def _sc_mesh():
    return plsc.VectorSubcoreMesh(core_axis_name="c", subcore_axis_name="s")


def _row_partition(N):
    rps = (N // 8 // N_SUBCORES) * 8
    tail = N - N_SUBCORES * rps
    return rps, tail


def _striped_rows_copy(src, dst, s, N):
    rps, tail = _row_partition(N)
    pltpu.sync_copy(src.at[pl.ds(s * rps, rps)], dst.at[pl.ds(s * rps, rps)])
    if tail:
        @pl.when(s == N_SUBCORES - 1)
        def _():
            pltpu.sync_copy(src.at[pl.ds(N_SUBCORES * rps, tail)],
                            dst.at[pl.ds(N_SUBCORES * rps, tail)])


NBUF = 5


def _make_deg_kernel(E, N):
    ept = E // N_TILES
    nch = ept // K

    @functools.partial(
        pl.kernel,
        out_type=jax.ShapeDtypeStruct((N_CORES, N, 16), jnp.float32),
        mesh=_sc_mesh(),
        compiler_params=pltpu.CompilerParams(use_tc_tiling_on_sc=False),
        scratch_types=[
            pltpu.VMEM((E // K // N_TILES, K), jnp.int32),
            pltpu.VMEM((K, 16), jnp.float32),
            pltpu.VMEM_SHARED((N, 16), jnp.float32),
            pltpu.SemaphoreType.DMA,
        ],
    )
    def deg_kernel(ei2_hbm, zero_hbm, ones_hbm, out_hbm,
                   didx, ones_v, acc_sh, ssem):
        c = lax.axis_index("c")
        s = lax.axis_index("s")
        wid = c * N_SUBCORES + s
        pltpu.sync_copy(ones_hbm, ones_v)
        pltpu.sync_copy(ei2_hbm.at[1].at[pl.ds(wid * nch, nch)], didx)
        _striped_rows_copy(zero_hbm, acc_sh, s, N)
        plsc.subcore_barrier()

        @pl.loop(0, nch, step=NBUF)
        def _(c0):
            for b in range(NBUF):
                pltpu.async_copy(ones_v, acc_sh.at[didx.at[c0 + b]], ssem,
                                 add=True)
            for b in range(NBUF):
                pltpu.make_async_copy(ones_v, acc_sh.at[didx.at[0]],
                                      ssem).wait()

        plsc.subcore_barrier()
        _striped_rows_copy(acc_sh, out_hbm.at[c], s, N)

    return deg_kernel


def _make_agg_kernel(E, N, D, feature_split):
    n_edge_workers = N_SUBCORES if feature_split else N_TILES
    ept = E // n_edge_workers
    nch = ept // K
    assert nch % NBUF == 0 and ept % K == 0
    g_shape = (N_CORES, N, D) if feature_split else (N, D)

    @functools.partial(
        pl.kernel,
        out_type=jax.ShapeDtypeStruct((N_CORES, N, D), jnp.float32),
        mesh=_sc_mesh(),
        compiler_params=pltpu.CompilerParams(use_tc_tiling_on_sc=False),
        scratch_types=[
            pltpu.VMEM((nch, K), jnp.int32),
            pltpu.VMEM((nch, K), jnp.int32),
            pltpu.VMEM((NBUF, K, D), jnp.float32),
            pltpu.VMEM_SHARED((N, D), jnp.float32),
            pltpu.SemaphoreType.DMA((NBUF,)),
            pltpu.SemaphoreType.DMA((NBUF,)),
        ],
    )
    def agg_kernel(g_hbm, ei2_hbm, zero_hbm, out_hbm,
                   sidx, didx, rows, acc_sh, gsem, ssem):
        c = lax.axis_index("c")
        s = lax.axis_index("s")
        wid = s if feature_split else c * N_SUBCORES + s
        gsrc = g_hbm.at[c] if feature_split else g_hbm
        pltpu.sync_copy(ei2_hbm.at[0].at[pl.ds(wid * nch, nch)], sidx)
        pltpu.sync_copy(ei2_hbm.at[1].at[pl.ds(wid * nch, nch)], didx)
        _striped_rows_copy(zero_hbm, acc_sh, s, N)
        plsc.subcore_barrier()

        @pl.loop(0, nch, step=NBUF)
        def _(c0):
            for b in range(NBUF):
                @pl.when(c0 > 0)
                def _():
                    pltpu.make_async_copy(rows.at[b],
                                          acc_sh.at[didx.at[0]],
                                          ssem.at[b]).wait()
                pltpu.async_copy(gsrc.at[sidx.at[c0 + b]], rows.at[b],
                                 gsem.at[b])
            for b in range(NBUF):
                pltpu.make_async_copy(gsrc.at[sidx.at[0]], rows.at[b],
                                      gsem.at[b]).wait()
                pltpu.async_copy(rows.at[b], acc_sh.at[didx.at[c0 + b]],
                                 ssem.at[b], add=True)

        for b in range(NBUF):
            pltpu.make_async_copy(rows.at[b], acc_sh.at[didx.at[0]],
                                  ssem.at[b]).wait()
        plsc.subcore_barrier()
        _striped_rows_copy(acc_sh, out_hbm.at[c], s, N)

    return agg_kernel


def _dot(a, b):
    return lax.dot_general(a, b, (((1,), (0,)), ((), ())),
                           precision=lax.Precision.HIGHEST,
                           preferred_element_type=jnp.float32)


def _mm_scale_body(x_ref, w_ref, deg_ref, g_ref, dinv_ref):
    degsum = deg_ref[0] + deg_ref[1]
    dinv = lax.rsqrt(degsum[:, 0:1] + 1.0)
    dinv_ref[...] = dinv
    g = _dot(x_ref[...], w_ref[...]) * dinv
    half = g.shape[1] // 2
    g_ref[0] = g[:, :half]
    g_ref[1] = g[:, half:]


def _epi1_body(agg_ref, g1_ref, dinv_ref, b1_ref, w2_ref, g2_ref):
    dinv = dinv_ref[...]
    summed = jnp.concatenate([agg_ref[0] + g1_ref[0],
                              agg_ref[1] + g1_ref[1]], axis=1)
    pre = dinv * summed + b1_ref[...]
    r = jnp.maximum(pre, 0.0)
    g2_ref[...] = _dot(r, w2_ref[...]) * dinv


def _epi2_body(agg_ref, g2_ref, dinv_ref, b2_ref, o_ref):
    o_ref[...] = (dinv_ref[...] * (agg_ref[0] + agg_ref[1] + g2_ref[...])
                  + b2_ref[...])


def kernel(x, edge_index, W1, b1, W2, b2):
    N, FEAT = x.shape
    HID = W1.shape[1]
    OUT = W2.shape[1]
    E = edge_index.shape[1]
    grid = (N // M_BLK,)

    HALF = HID // N_CORES
    ei2 = edge_index.reshape(2, E // K, K)
    z16 = jnp.zeros((N, 16), jnp.float32)
    ones = jnp.ones((K, 16), jnp.float32)
    zH = jnp.zeros((N, HALF), jnp.float32)
    zO = jnp.zeros((N, OUT), jnp.float32)

    deg = _make_deg_kernel(E, N)(ei2, z16, ones)

    g1, dinv = pl.pallas_call(
        _mm_scale_body,
        grid=grid,
        in_specs=[pl.BlockSpec((M_BLK, FEAT), lambda i: (i, 0)),
                  pl.BlockSpec((FEAT, HID), lambda i: (0, 0)),
                  pl.BlockSpec((N_CORES, M_BLK, 16), lambda i: (0, i, 0))],
        out_specs=[pl.BlockSpec((N_CORES, M_BLK, HALF), lambda i: (0, i, 0)),
                   pl.BlockSpec((M_BLK, 1), lambda i: (i, 0))],
        out_shape=[jax.ShapeDtypeStruct((N_CORES, N, HALF), jnp.float32),
                   jax.ShapeDtypeStruct((N, 1), jnp.float32)],
    )(x, W1, deg)

    agg1 = _make_agg_kernel(E, N, HALF, True)(g1, ei2, zH)

    g2 = pl.pallas_call(
        _epi1_body,
        grid=grid,
        in_specs=[pl.BlockSpec((N_CORES, M_BLK, HALF), lambda i: (0, i, 0)),
                  pl.BlockSpec((N_CORES, M_BLK, HALF), lambda i: (0, i, 0)),
                  pl.BlockSpec((M_BLK, 1), lambda i: (i, 0)),
                  pl.BlockSpec((1, HID), lambda i: (0, 0)),
                  pl.BlockSpec((HID, OUT), lambda i: (0, 0))],
        out_specs=pl.BlockSpec((M_BLK, OUT), lambda i: (i, 0)),
        out_shape=jax.ShapeDtypeStruct((N, OUT), jnp.float32),
    )(agg1, g1, dinv, b1.reshape(1, HID), W2)

    agg2 = _make_agg_kernel(E, N, OUT, False)(g2, ei2, zO)

    out = pl.pallas_call(
        _epi2_body,
        grid=grid,
        in_specs=[pl.BlockSpec((N_CORES, M_BLK, OUT), lambda i: (0, i, 0)),
                  pl.BlockSpec((M_BLK, OUT), lambda i: (i, 0)),
                  pl.BlockSpec((M_BLK, 1), lambda i: (i, 0)),
                  pl.BlockSpec((1, OUT), lambda i: (0, 0))],
        out_specs=pl.BlockSpec((M_BLK, OUT), lambda i: (i, 0)),
        out_shape=jax.ShapeDtypeStruct((N, OUT), jnp.float32),
    )(agg2, g2, dinv, b2.reshape(1, OUT))

    return out
